# Initial kernel scaffold; baseline (speedup 1.0000x reference)
import jax
import jax.numpy as jnp
from jax.experimental import pallas as pl


def kernel(node_features, edge_features, edge_index, W1, b1, W2, b2):
    raise NotImplementedError("write your pallas kernel here")


# speedup vs baseline: 2.6498x; 2.6498x over previous
import functools

import jax
import jax.numpy as jnp
from jax import lax
from jax.experimental import pallas as pl
from jax.experimental.pallas import tpu as pltpu
from jax.experimental.pallas import tpu_sc as plsc

N_NODES = 10000
N_EDGES = 320000
NODE_DIM = 128
EDGE_DIM = 16
HIDDEN_DIM = 32

NC = 2
NS = 16
NW = NC * NS
BPW = N_EDGES // NW
CH = 80
NCHUNK = BPW // CH


def _proj_body(nf_ref, wa_ref, wb_ref, pa_ref, pb_ref):
    nf = nf_ref[...]
    pa_ref[...] = jnp.dot(nf, wa_ref[...], preferred_element_type=jnp.float32)
    pb_ref[...] = jnp.dot(nf, wb_ref[...], preferred_element_type=jnp.float32)


_proj_call = pl.pallas_call(
    _proj_body,
    out_shape=[
        jax.ShapeDtypeStruct((N_NODES, HIDDEN_DIM), jnp.float32),
        jax.ShapeDtypeStruct((N_NODES, HIDDEN_DIM), jnp.float32),
    ],
)


_mesh = plsc.VectorSubcoreMesh(
    core_axis_name="c", subcore_axis_name="s", num_cores=NC, num_subcores=NS
)


@functools.partial(
    pl.kernel,
    out_type=jax.ShapeDtypeStruct((N_EDGES, HIDDEN_DIM), jnp.float32),
    mesh=_mesh,
    scratch_types=[
        pltpu.VMEM((CH,), jnp.int32),
        pltpu.VMEM((CH,), jnp.int32),
        pltpu.VMEM((CH, HIDDEN_DIM), jnp.float32),
        pltpu.VMEM((CH, HIDDEN_DIM), jnp.float32),
        pltpu.VMEM((CH, HIDDEN_DIM), jnp.float32),
        pltpu.SemaphoreType.DMA,
        pltpu.SemaphoreType.DMA,
    ],
    compiler_params=pltpu.CompilerParams(use_tc_tiling_on_sc=False),
)
def _gather_add(pa_hbm, pb_hbm, src_hbm, dst_hbm, out_hbm,
                src_v, dst_v, a_v, b_v, o_v, sem_a, sem_b):
    wid = lax.axis_index("s") * NC + lax.axis_index("c")
    base = wid * BPW

    def chunk(i, carry):
        off = base + i * CH
        pltpu.sync_copy(src_hbm.at[pl.ds(off, CH)], src_v)
        pltpu.sync_copy(dst_hbm.at[pl.ds(off, CH)], dst_v)
        ca = pltpu.async_copy(pa_hbm.at[src_v], a_v, sem_a)
        cb = pltpu.async_copy(pb_hbm.at[dst_v], b_v, sem_b)
        ca.wait()
        cb.wait()
        for j in range(CH):
            o_v[j, 0:16] = a_v[j, 0:16] + b_v[j, 0:16]
            o_v[j, 16:32] = a_v[j, 16:32] + b_v[j, 16:32]
        pltpu.sync_copy(o_v, out_hbm.at[pl.ds(off, CH)])
        return carry

    lax.fori_loop(0, NCHUNK, chunk, 0)


_ROWS = 6400
_NBLK = N_EDGES // _ROWS


def _mlp_body(g_ref, e_ref, w1e_ref, b1_ref, w2_ref, b2_ref, out_ref):
    e = e_ref[...]
    pre = g_ref[...] + jnp.dot(e, w1e_ref[...],
                               preferred_element_type=jnp.float32) + b1_ref[...]
    h = jnp.maximum(pre, 0.0)
    out_ref[...] = e + jnp.dot(h, w2_ref[...],
                               preferred_element_type=jnp.float32) + b2_ref[...]


_mlp_call = pl.pallas_call(
    _mlp_body,
    grid=(_NBLK,),
    in_specs=[
        pl.BlockSpec((_ROWS, HIDDEN_DIM), lambda i: (i, 0)),
        pl.BlockSpec((_ROWS, EDGE_DIM), lambda i: (i, 0)),
        pl.BlockSpec((EDGE_DIM, HIDDEN_DIM), lambda i: (0, 0)),
        pl.BlockSpec((1, HIDDEN_DIM), lambda i: (0, 0)),
        pl.BlockSpec((HIDDEN_DIM, EDGE_DIM), lambda i: (0, 0)),
        pl.BlockSpec((1, EDGE_DIM), lambda i: (0, 0)),
    ],
    out_specs=pl.BlockSpec((_ROWS, EDGE_DIM), lambda i: (i, 0)),
    out_shape=jax.ShapeDtypeStruct((N_EDGES, EDGE_DIM), jnp.float32),
)


def kernel(node_features, edge_features, edge_index, W1, b1, W2, b2):
    src = edge_index[0].astype(jnp.int32)
    dst = edge_index[1].astype(jnp.int32)
    pa, pb = _proj_call(node_features, W1[:NODE_DIM], W1[NODE_DIM:2 * NODE_DIM])
    g = _gather_add(pa, pb, src, dst)
    return _mlp_call(g, edge_features, W1[2 * NODE_DIM:],
                     b1.reshape(1, HIDDEN_DIM), W2, b2.reshape(1, EDGE_DIM))


# speedup vs baseline: 3.5970x; 1.3575x over previous
import functools

import jax
import jax.numpy as jnp
from jax import lax
from jax.experimental import pallas as pl
from jax.experimental.pallas import tpu as pltpu
from jax.experimental.pallas import tpu_sc as plsc

N_NODES = 10000
N_EDGES = 320000
NODE_DIM = 128
EDGE_DIM = 16
HIDDEN_DIM = 32

NC = 2
NS = 16
NW = NC * NS
BPW = N_EDGES // NW
CH = 80
NCHUNK = BPW // CH


def _proj_body(nf_ref, wa_ref, wb_ref, pa_ref, pb_ref):
    nf = nf_ref[...]
    pa_ref[...] = jnp.dot(nf, wa_ref[...], preferred_element_type=jnp.float32)
    pb_ref[...] = jnp.dot(nf, wb_ref[...], preferred_element_type=jnp.float32)


_proj_call = pl.pallas_call(
    _proj_body,
    out_shape=[
        jax.ShapeDtypeStruct((N_NODES, HIDDEN_DIM), jnp.float32),
        jax.ShapeDtypeStruct((N_NODES, HIDDEN_DIM), jnp.float32),
    ],
)


_mesh = plsc.VectorSubcoreMesh(
    core_axis_name="c", subcore_axis_name="s", num_cores=NC, num_subcores=NS
)


NBUF = 5
NOUTER = NCHUNK // NBUF


@functools.partial(
    pl.kernel,
    out_type=jax.ShapeDtypeStruct((N_EDGES, HIDDEN_DIM), jnp.float32),
    mesh=_mesh,
    scratch_types=[
        pltpu.VMEM((BPW,), jnp.int32),
        pltpu.VMEM((BPW,), jnp.int32),
        pltpu.VMEM((NBUF, CH, HIDDEN_DIM), jnp.float32),
        pltpu.VMEM((NBUF, CH, HIDDEN_DIM), jnp.float32),
        pltpu.VMEM((NBUF, CH, HIDDEN_DIM), jnp.float32),
        [pltpu.SemaphoreType.DMA] * NBUF,
        [pltpu.SemaphoreType.DMA] * NBUF,
    ],
    compiler_params=pltpu.CompilerParams(use_tc_tiling_on_sc=False),
)
def _gather_add(pa_hbm, pb_hbm, src_hbm, dst_hbm, out_hbm,
                src_all, dst_all, a_v, b_v, o_v, gsems, ssems):
    wid = lax.axis_index("s") * NC + lax.axis_index("c")
    base = wid * BPW

    pltpu.sync_copy(src_hbm.at[pl.ds(base, BPW)], src_all)
    pltpu.sync_copy(dst_hbm.at[pl.ds(base, BPW)], dst_all)

    def gathers(i, b):
        loc = i * CH
        ca = pltpu.make_async_copy(
            pa_hbm.at[src_all.at[pl.ds(loc, CH)]], a_v.at[b], gsems[b])
        cb = pltpu.make_async_copy(
            pb_hbm.at[dst_all.at[pl.ds(loc, CH)]], b_v.at[b], gsems[b])
        return ca, cb

    def store(i, b):
        return pltpu.make_async_copy(
            o_v.at[b], out_hbm.at[pl.ds(base + i * CH, CH)], ssems[b])

    for b in range(NBUF):
        ca, cb = gathers(b, b)
        ca.start()
        cb.start()

    def outer(t, carry):
        for b in range(NBUF):
            i = t * NBUF + b
            ca, cb = gathers(i, b)
            ca.wait()
            cb.wait()

            @pl.when(t > 0)
            def _():
                store(i - NBUF, b).wait()

            for j in range(CH):
                o_v[b, j, 0:16] = a_v[b, j, 0:16] + b_v[b, j, 0:16]
                o_v[b, j, 16:32] = a_v[b, j, 16:32] + b_v[b, j, 16:32]
            store(i, b).start()

            @pl.when(t < NOUTER - 1)
            def _():
                na, nb = gathers(i + NBUF, b)
                na.start()
                nb.start()

        return carry

    lax.fori_loop(0, NOUTER, outer, 0)

    for b in range(NBUF):
        store((NOUTER - 1) * NBUF + b, b).wait()


_ROWS = 6400
_NBLK = N_EDGES // _ROWS


def _mlp_body(g_ref, e_ref, w1e_ref, b1_ref, w2_ref, b2_ref, out_ref):
    e = e_ref[...]
    pre = g_ref[...] + jnp.dot(e, w1e_ref[...],
                               preferred_element_type=jnp.float32) + b1_ref[...]
    h = jnp.maximum(pre, 0.0)
    out_ref[...] = e + jnp.dot(h, w2_ref[...],
                               preferred_element_type=jnp.float32) + b2_ref[...]


_mlp_call = pl.pallas_call(
    _mlp_body,
    grid=(_NBLK,),
    in_specs=[
        pl.BlockSpec((_ROWS, HIDDEN_DIM), lambda i: (i, 0)),
        pl.BlockSpec((_ROWS, EDGE_DIM), lambda i: (i, 0)),
        pl.BlockSpec((EDGE_DIM, HIDDEN_DIM), lambda i: (0, 0)),
        pl.BlockSpec((1, HIDDEN_DIM), lambda i: (0, 0)),
        pl.BlockSpec((HIDDEN_DIM, EDGE_DIM), lambda i: (0, 0)),
        pl.BlockSpec((1, EDGE_DIM), lambda i: (0, 0)),
    ],
    out_specs=pl.BlockSpec((_ROWS, EDGE_DIM), lambda i: (i, 0)),
    out_shape=jax.ShapeDtypeStruct((N_EDGES, EDGE_DIM), jnp.float32),
)


def kernel(node_features, edge_features, edge_index, W1, b1, W2, b2):
    src = edge_index[0].astype(jnp.int32)
    dst = edge_index[1].astype(jnp.int32)
    pa, pb = _proj_call(node_features, W1[:NODE_DIM], W1[NODE_DIM:2 * NODE_DIM])
    g = _gather_add(pa, pb, src, dst)
    return _mlp_call(g, edge_features, W1[2 * NODE_DIM:],
                     b1.reshape(1, HIDDEN_DIM), W2, b2.reshape(1, EDGE_DIM))


# speedup vs baseline: 4.4874x; 1.2475x over previous
import functools

import jax
import jax.numpy as jnp
from jax import lax
from jax.experimental import pallas as pl
from jax.experimental.pallas import tpu as pltpu
from jax.experimental.pallas import tpu_sc as plsc

N_NODES = 10000
N_EDGES = 320000
NODE_DIM = 128
EDGE_DIM = 16
HIDDEN_DIM = 32

NC = 2
NS = 16
NW = NC * NS
BPW = N_EDGES // NW
CH = 80
NCHUNK = BPW // CH
NBUF = 5
NOUTER = NCHUNK // NBUF

G_COLS = 8 * HIDDEN_DIM
G_ROWS = N_EDGES * HIDDEN_DIM // G_COLS
CH_GROWS = CH * HIDDEN_DIM // G_COLS
BPW_GROWS = BPW * HIDDEN_DIM // G_COLS

E_COLS = 8 * EDGE_DIM
E_ROWS = N_EDGES // 8


def _proj_body(nf_ref, wa_ref, wb_ref, pa_ref, pb_ref):
    nf = nf_ref[...]
    pa_ref[...] = jnp.dot(nf, wa_ref[...], preferred_element_type=jnp.float32)
    pb_ref[...] = jnp.dot(nf, wb_ref[...], preferred_element_type=jnp.float32)


_proj_call = pl.pallas_call(
    _proj_body,
    out_shape=[
        jax.ShapeDtypeStruct((N_NODES, HIDDEN_DIM), jnp.float32),
        jax.ShapeDtypeStruct((N_NODES, HIDDEN_DIM), jnp.float32),
    ],
)


_mesh = plsc.VectorSubcoreMesh(
    core_axis_name="c", subcore_axis_name="s", num_cores=NC, num_subcores=NS
)


@functools.partial(
    pl.kernel,
    out_type=jax.ShapeDtypeStruct((G_ROWS, G_COLS), jnp.float32),
    mesh=_mesh,
    scratch_types=[
        pltpu.VMEM((BPW,), jnp.int32),
        pltpu.VMEM((BPW,), jnp.int32),
        pltpu.VMEM((NBUF, CH, HIDDEN_DIM), jnp.float32),
        pltpu.VMEM((NBUF, CH, HIDDEN_DIM), jnp.float32),
        pltpu.VMEM((NBUF, CH_GROWS, G_COLS), jnp.float32),
        [pltpu.SemaphoreType.DMA] * NBUF,
        [pltpu.SemaphoreType.DMA] * NBUF,
    ],
    compiler_params=pltpu.CompilerParams(use_tc_tiling_on_sc=False),
)
def _gather_add(pa_hbm, pb_hbm, src_hbm, dst_hbm, out_hbm,
                src_all, dst_all, a_v, b_v, o_v, gsems, ssems):
    wid = lax.axis_index("s") * NC + lax.axis_index("c")
    base = wid * BPW
    gbase = wid * BPW_GROWS

    pltpu.sync_copy(src_hbm.at[pl.ds(base, BPW)], src_all)
    pltpu.sync_copy(dst_hbm.at[pl.ds(base, BPW)], dst_all)

    def gathers(i, b):
        loc = i * CH
        ca = pltpu.make_async_copy(
            pa_hbm.at[src_all.at[pl.ds(loc, CH)]], a_v.at[b], gsems[b])
        cb = pltpu.make_async_copy(
            pb_hbm.at[dst_all.at[pl.ds(loc, CH)]], b_v.at[b], gsems[b])
        return ca, cb

    def store(i, b):
        return pltpu.make_async_copy(
            o_v.at[b], out_hbm.at[pl.ds(gbase + i * CH_GROWS, CH_GROWS)],
            ssems[b])

    for b in range(NBUF):
        ca, cb = gathers(b, b)
        ca.start()
        cb.start()

    def outer(t, carry):
        for b in range(NBUF):
            i = t * NBUF + b
            ca, cb = gathers(i, b)
            ca.wait()
            cb.wait()

            @pl.when(t > 0)
            def _():
                store(i - NBUF, b).wait()

            for j in range(CH):
                r, c = j // 8, (j % 8) * HIDDEN_DIM
                o_v[b, r, c:c + 16] = a_v[b, j, 0:16] + b_v[b, j, 0:16]
                o_v[b, r, c + 16:c + 32] = a_v[b, j, 16:32] + b_v[b, j, 16:32]
            store(i, b).start()

            @pl.when(t < NOUTER - 1)
            def _():
                na, nb = gathers(i + NBUF, b)
                na.start()
                nb.start()

        return carry

    lax.fori_loop(0, NOUTER, outer, 0)

    for b in range(NBUF):
        store((NOUTER - 1) * NBUF + b, b).wait()


_ROWS = 2000
_NBLK = E_ROWS // _ROWS


def _mlp_body(g_ref, e_ref, w1e_ref, b1_ref, w2_ref, b2_ref, out_ref):
    e = e_ref[...]
    pre = g_ref[...] + jnp.dot(e, w1e_ref[...],
                               preferred_element_type=jnp.float32) + b1_ref[...]
    h = jnp.maximum(pre, 0.0)
    out_ref[...] = e + jnp.dot(h, w2_ref[...],
                               preferred_element_type=jnp.float32) + b2_ref[...]


_mlp_call = pl.pallas_call(
    _mlp_body,
    grid=(_NBLK,),
    in_specs=[
        pl.BlockSpec((_ROWS, G_COLS), lambda i: (i, 0)),
        pl.BlockSpec((_ROWS, E_COLS), lambda i: (i, 0)),
        pl.BlockSpec((E_COLS, G_COLS), lambda i: (0, 0)),
        pl.BlockSpec((1, G_COLS), lambda i: (0, 0)),
        pl.BlockSpec((G_COLS, E_COLS), lambda i: (0, 0)),
        pl.BlockSpec((1, E_COLS), lambda i: (0, 0)),
    ],
    out_specs=pl.BlockSpec((_ROWS, E_COLS), lambda i: (i, 0)),
    out_shape=jax.ShapeDtypeStruct((E_ROWS, E_COLS), jnp.float32),
)


def kernel(node_features, edge_features, edge_index, W1, b1, W2, b2):
    src = edge_index[0].astype(jnp.int32)
    dst = edge_index[1].astype(jnp.int32)
    pa, pb = _proj_call(node_features, W1[:NODE_DIM], W1[NODE_DIM:2 * NODE_DIM])
    g = _gather_add(pa, pb, src, dst)

    eye8 = jnp.eye(8, dtype=jnp.float32)
    w1e_blk = jnp.kron(eye8, W1[2 * NODE_DIM:])
    w2_blk = jnp.kron(eye8, W2)
    b1_blk = jnp.tile(b1, 8).reshape(1, G_COLS)
    b2_blk = jnp.tile(b2, 8).reshape(1, E_COLS)
    e128 = edge_features.reshape(E_ROWS, E_COLS)

    out = _mlp_call(g, e128, w1e_blk, b1_blk, w2_blk, b2_blk)
    return out.reshape(N_EDGES, EDGE_DIM)


# speedup vs baseline: 5.0206x; 1.1188x over previous
import functools

import jax
import jax.numpy as jnp
from jax import lax
from jax.experimental import pallas as pl
from jax.experimental.pallas import tpu as pltpu
from jax.experimental.pallas import tpu_sc as plsc

N_NODES = 10000
N_EDGES = 320000
NODE_DIM = 128
EDGE_DIM = 16
HIDDEN_DIM = 32

NC = 2
NS = 16
NW = NC * NS
BPW = N_EDGES // NW
CH = 80
NCHUNK = BPW // CH
NBUF = 5
NOUTER = NCHUNK // NBUF

G_COLS = 4 * HIDDEN_DIM
G_ROWS = N_EDGES * HIDDEN_DIM // G_COLS
CH_GROWS = CH * HIDDEN_DIM // G_COLS
BPW_GROWS = BPW * HIDDEN_DIM // G_COLS
H_COLS = 8 * HIDDEN_DIM

E_COLS = 8 * EDGE_DIM
E_ROWS = N_EDGES // 8


def _proj_body(nf_ref, wa_ref, wb_ref, pa_ref, pb_ref):
    nf = nf_ref[...]
    pa_ref[...] = jnp.dot(nf, wa_ref[...], preferred_element_type=jnp.float32)
    pb_ref[...] = jnp.dot(nf, wb_ref[...], preferred_element_type=jnp.float32)


_proj_call = pl.pallas_call(
    _proj_body,
    out_shape=[
        jax.ShapeDtypeStruct((N_NODES, HIDDEN_DIM), jnp.float32),
        jax.ShapeDtypeStruct((N_NODES, HIDDEN_DIM), jnp.float32),
    ],
)


_mesh = plsc.VectorSubcoreMesh(
    core_axis_name="c", subcore_axis_name="s", num_cores=NC, num_subcores=NS
)


@functools.partial(
    pl.kernel,
    out_type=jax.ShapeDtypeStruct((G_ROWS, G_COLS), jnp.float32),
    mesh=_mesh,
    scratch_types=[
        pltpu.VMEM((BPW,), jnp.int32),
        pltpu.VMEM((BPW,), jnp.int32),
        pltpu.VMEM((NBUF, CH, HIDDEN_DIM), jnp.float32),
        pltpu.VMEM((NBUF, CH, HIDDEN_DIM), jnp.float32),
        pltpu.VMEM((NBUF, CH_GROWS, G_COLS), jnp.float32),
        [pltpu.SemaphoreType.DMA] * NBUF,
        [pltpu.SemaphoreType.DMA] * NBUF,
    ],
    compiler_params=pltpu.CompilerParams(use_tc_tiling_on_sc=False),
)
def _gather_add(pa_hbm, pb_hbm, src_hbm, dst_hbm, out_hbm,
                src_all, dst_all, a_v, b_v, o_v, gsems, ssems):
    wid = lax.axis_index("s") * NC + lax.axis_index("c")
    base = wid * BPW
    gbase = wid * BPW_GROWS

    pltpu.sync_copy(src_hbm.at[pl.ds(base, BPW)], src_all)
    pltpu.sync_copy(dst_hbm.at[pl.ds(base, BPW)], dst_all)

    def gathers(i, b):
        loc = i * CH
        ca = pltpu.make_async_copy(
            pa_hbm.at[src_all.at[pl.ds(loc, CH)]], a_v.at[b], gsems[b])
        cb = pltpu.make_async_copy(
            pb_hbm.at[dst_all.at[pl.ds(loc, CH)]], b_v.at[b], gsems[b])
        return ca, cb

    def store(i, b):
        return pltpu.make_async_copy(
            o_v.at[b], out_hbm.at[pl.ds(gbase + i * CH_GROWS, CH_GROWS)],
            ssems[b])

    for b in range(NBUF):
        ca, cb = gathers(b, b)
        ca.start()
        cb.start()

    def outer(t, carry):
        for b in range(NBUF):
            i = t * NBUF + b
            ca, cb = gathers(i, b)
            ca.wait()
            cb.wait()

            @pl.when(t > 0)
            def _():
                store(i - NBUF, b).wait()

            for j in range(CH):
                r, c = j // 4, (j % 4) * HIDDEN_DIM
                o_v[b, r, c:c + 16] = a_v[b, j, 0:16] + b_v[b, j, 0:16]
                o_v[b, r, c + 16:c + 32] = a_v[b, j, 16:32] + b_v[b, j, 16:32]
            store(i, b).start()

            @pl.when(t < NOUTER - 1)
            def _():
                na, nb = gathers(i + NBUF, b)
                na.start()
                nb.start()

        return carry

    lax.fori_loop(0, NOUTER, outer, 0)

    for b in range(NBUF):
        store((NOUTER - 1) * NBUF + b, b).wait()


_ROWS = 2000
_NBLK = E_ROWS // _ROWS


def _mlp_body(g_ref, e_ref, w1e_ref, b1_ref, w2_ref, b2_ref, out_ref):
    e = e_ref[...]
    g = g_ref[...].reshape(_ROWS, H_COLS)
    pre = g + jnp.dot(e, w1e_ref[...],
                      preferred_element_type=jnp.float32) + b1_ref[...]
    h = jnp.maximum(pre, 0.0)
    out_ref[...] = e + jnp.dot(h, w2_ref[...],
                               preferred_element_type=jnp.float32) + b2_ref[...]


_mlp_call = pl.pallas_call(
    _mlp_body,
    grid=(_NBLK,),
    in_specs=[
        pl.BlockSpec((2 * _ROWS, G_COLS), lambda i: (i, 0)),
        pl.BlockSpec((_ROWS, E_COLS), lambda i: (i, 0)),
        pl.BlockSpec((E_COLS, H_COLS), lambda i: (0, 0)),
        pl.BlockSpec((1, H_COLS), lambda i: (0, 0)),
        pl.BlockSpec((H_COLS, E_COLS), lambda i: (0, 0)),
        pl.BlockSpec((1, E_COLS), lambda i: (0, 0)),
    ],
    out_specs=pl.BlockSpec((_ROWS, E_COLS), lambda i: (i, 0)),
    out_shape=jax.ShapeDtypeStruct((E_ROWS, E_COLS), jnp.float32),
)


def kernel(node_features, edge_features, edge_index, W1, b1, W2, b2):
    src = edge_index[0].astype(jnp.int32)
    dst = edge_index[1].astype(jnp.int32)
    pa, pb = _proj_call(node_features, W1[:NODE_DIM], W1[NODE_DIM:2 * NODE_DIM])
    g = _gather_add(pa, pb, src, dst)

    eye8 = jnp.eye(8, dtype=jnp.float32)
    w1e_blk = jnp.kron(eye8, W1[2 * NODE_DIM:])
    w2_blk = jnp.kron(eye8, W2)
    b1_blk = jnp.tile(b1, 8).reshape(1, H_COLS)
    b2_blk = jnp.tile(b2, 8).reshape(1, E_COLS)
    e128 = edge_features.reshape(E_ROWS, E_COLS)

    out = _mlp_call(g, e128, w1e_blk, b1_blk, w2_blk, b2_blk)
    return out.reshape(N_EDGES, EDGE_DIM)
